# baseline (speedup 1.0000x reference)
import functools

import jax
import jax.numpy as jnp
from jax import lax
from jax.experimental import pallas as pl
from jax.experimental.pallas import tpu as pltpu
from jax.experimental.pallas import tpu_sc as plsc

N = 10000
NP = 10240
IN_F = 5
HID = 256
K = 122
NG = 128
NGP = 256
E = 320000
ITEMS = E + N
NCORE = 2
NSUB = 16
NW = NCORE * NSUB
CB = 128
NCH = 88
GRP = 8
NGRPS = NCH // GRP
TPW = NCH * CB
EPAD = TPW * NW
EROWS = EPAD // CB
NPT = NP // NSUB
C0SZ = NG * NP
C0PT = C0SZ // NSUB
HIGH = lax.Precision.HIGHEST


def _hist_body(dst_ref, bhist_ref, deg_out, cnt_out,
               buf_v, hist_v, bbuf_v, chist_v, red_v, out_v, cred_v, cout_v,
               deg_sh, cnt_sh):
    core = lax.axis_index("c")
    sid = lax.axis_index("s")
    wid = sid * NCORE + core
    z16 = jnp.zeros((16,), jnp.float32)
    o16 = jnp.ones((16,), jnp.float32)

    def zero_hist(i, c):
        hist_v[pl.ds(i * 16, 16)] = z16
        return c
    lax.fori_loop(0, NP // 16, zero_hist, 0)

    def zero_chist(i, c):
        chist_v[pl.ds(i * 16, 16)] = z16
        return c
    lax.fori_loop(0, NGP // 16, zero_chist, 0)

    pltpu.sync_copy(dst_ref.at[pl.ds(wid * TPW, TPW)], buf_v)

    def scat(i, c):
        idx = buf_v[pl.ds(i * 16, 16)]
        plsc.addupdate_scatter(hist_v, [idx], o16)
        return c
    lax.fori_loop(0, TPW // 16, scat, 0)

    bpt = NP // NW
    pltpu.sync_copy(bhist_ref.at[pl.ds(wid * bpt, bpt)], bbuf_v)

    def bscat(i, c):
        idx = bbuf_v[pl.ds(i * 16, 16)]
        plsc.addupdate_scatter(chist_v, [idx], o16)
        return c
    lax.fori_loop(0, bpt // 16, bscat, 0)

    pltpu.sync_copy(hist_v, deg_sh.at[sid])
    pltpu.sync_copy(chist_v, cnt_sh.at[sid])
    plsc.subcore_barrier()

    for r in range(NSUB):
        pltpu.sync_copy(deg_sh.at[r, pl.ds(sid * NPT, NPT)], red_v.at[r])

    def red(i, c):
        s = red_v[0, pl.ds(i * 16, 16)]
        for r in range(1, NSUB):
            s = s + red_v[r, pl.ds(i * 16, 16)]
        out_v[pl.ds(i * 16, 16)] = s
        return c
    lax.fori_loop(0, NPT // 16, red, 0)
    pltpu.sync_copy(out_v, deg_out.at[core, pl.ds(sid * NPT, NPT)])

    @pl.when(sid == 0)
    def _():
        pltpu.sync_copy(cnt_sh, cred_v)

        def cred(i, c):
            s = cred_v[0, pl.ds(i * 16, 16)]
            for r in range(1, NSUB):
                s = s + cred_v[r, pl.ds(i * 16, 16)]
            cout_v[pl.ds(i * 16, 16)] = s
            return c
        lax.fori_loop(0, NGP // 16, cred, 0)
        pltpu.sync_copy(cout_v, cnt_out.at[core])


def _edge_body(items_ref, y_ref, ptab_ref, zacc_ref, zc0_ref,
               acc_out, c0_out,
               ebuf_v, ptab_v, msg_v, wgth_v, flat_v,
               acc_sh, c0_sh, sem_g, sem_s):
    core = lax.axis_index("c")
    sid = lax.axis_index("s")
    wid = sid * NCORE + core

    pltpu.sync_copy(zacc_ref, acc_sh.at[pl.ds(sid * NPT, NPT)])
    pltpu.sync_copy(zc0_ref, c0_sh.at[pl.ds(sid * C0PT, C0PT)])

    pltpu.sync_copy(ptab_ref, ptab_v)
    plsc.subcore_barrier()

    def fire_gathers(slot, grp):
        hs = []
        for b in range(GRP):
            hs.append(pltpu.async_copy(
                y_ref.at[ebuf_v.at[slot, b, 0]], msg_v.at[slot, b], sem_g))
        return hs

    def load_idx(slot, grp):
        pltpu.sync_copy(items_ref.at[pl.ds(wid * NCH + grp * GRP, GRP)],
                        ebuf_v.at[slot])

    load_idx(0, 0)
    fire_gathers(0, 0)

    def group(g, carry):
        slot = lax.rem(g, 2)
        slot2 = 1 - slot
        for b in range(GRP):
            pltpu.make_async_copy(
                y_ref.at[ebuf_v.at[slot, b, 0]], msg_v.at[slot, b],
                sem_g).wait()
        gnext = jnp.minimum(g + 1, NGRPS - 1)
        load_idx(slot2, gnext)
        fire_gathers(slot2, gnext)
        for b in range(GRP):
            for j in range(CB // 16):
                d16 = ebuf_v[slot, b, 1, pl.ds(j * 16, 16)]
                s16 = ebuf_v[slot, b, 0, pl.ds(j * 16, 16)]
                word = plsc.load_gather(ptab_v, [d16])
                flat_v[slot, b, pl.ds(j * 16, 16)] = (
                    (word & jnp.int32(511)) * NP + s16)
                wgth_v[slot, b, pl.ds(j * 16, 16)] = plsc.bitcast(
                    word & jnp.int32(-512), jnp.float32)
        sh = []
        for b in range(GRP):
            sh.append(pltpu.async_copy(
                msg_v.at[slot, b], acc_sh.at[ebuf_v.at[slot, b, 1]],
                sem_s, add=True))
            sh.append(pltpu.async_copy(
                wgth_v.at[slot, b], c0_sh.at[flat_v.at[slot, b]],
                sem_s, add=True))
        for h in sh:
            h.wait()
        return carry
    lax.fori_loop(0, NGRPS, group, 0)

    lastslot = NGRPS % 2
    for b in range(GRP):
        pltpu.make_async_copy(
            y_ref.at[ebuf_v.at[lastslot, b, 0]], msg_v.at[lastslot, b],
            sem_g).wait()

    plsc.subcore_barrier()
    pltpu.sync_copy(acc_sh.at[pl.ds(sid * NPT, NPT)],
                    acc_out.at[core, pl.ds(sid * NPT, NPT)])
    pltpu.sync_copy(c0_sh.at[pl.ds(sid * C0PT, C0PT)],
                    c0_out.at[core, pl.ds(sid * C0PT, C0PT)])


KB1 = 2048
NB1 = NP // KB1


def _norm_body(degp, cntp, batch2, xp, dinv_o, ptab_o, y_o, occ_o):
    j = pl.program_id(0)
    deg = degp[0, :] + degp[1, :]
    dinv = jnp.where(deg > 0, lax.rsqrt(deg), 0.0).reshape(KB1, 1)
    cnt = cntp[0, :NG] + cntp[1, :NG]
    invc = 1.0 / jnp.maximum(cnt, 1.0)
    occ_o[...] = (cnt > 0).astype(jnp.float32).reshape(NG, 1)
    b = batch2[...]
    onehot = (b == lax.broadcasted_iota(jnp.int32, (KB1, NG), 1))
    invn = jnp.sum(jnp.where(onehot, invc.reshape(1, NG), 0.0),
                   axis=1, keepdims=True)
    nid = j * KB1 + lax.broadcasted_iota(jnp.int32, (KB1, 1), 0)
    dinv_o[...] = dinv
    wdst = jnp.where(nid < N, dinv * invn, 0.0)
    bits = lax.bitcast_convert_type(wdst, jnp.int32)
    ptab_o[...] = ((bits + 256) & jnp.int32(-512)) | b
    y_o[...] = xp[...] * dinv


KB2 = 2048
NB2 = NP // KB2


def _dense_body(accp, dinv_ref, c0_ref, w1_ref, b1_ref, w2_ref, wl_ref,
                b2_ref, bl_ref, occ_ref, out_ref, m_ref):
    j = pl.program_id(0)

    @pl.when(j == 0)
    def _():
        m_ref[...] = jnp.zeros_like(m_ref)

    acc = accp[0] + accp[1]
    dinv = dinv_ref[...]
    h1 = jnp.dot(acc * dinv, w1_ref[...], precision=HIGH) + b1_ref[...]
    r = jnp.maximum(h1, 0.0)
    nid = j * KB2 + lax.broadcasted_iota(jnp.int32, (KB2, 1), 0)
    rd = jnp.where(nid < N, r * dinv, 0.0)
    c0b = c0_ref[0] + c0_ref[1]
    m_ref[...] += jnp.dot(c0b, rd, precision=HIGH)

    @pl.when(j == NB2 - 1)
    def _():
        w2l = jnp.dot(w2_ref[...], wl_ref[...], precision=HIGH)
        bl2 = jnp.dot(b2_ref[...], wl_ref[...], precision=HIGH)
        out_ref[...] = (jnp.dot(m_ref[...], w2l, precision=HIGH)
                        + occ_ref[...] * bl2 + bl_ref[...])


def kernel(x, edge_index, batch, W1, b1, W2, b2, Wlin, blin):
    f32 = jnp.float32
    ei = edge_index.astype(jnp.int32)
    bt = batch.astype(jnp.int32)
    loops = jnp.arange(N, dtype=jnp.int32)
    pad = jnp.full((EPAD - ITEMS,), N, jnp.int32)
    src_flat = jnp.concatenate([ei[0], loops, pad])
    dst_flat = jnp.concatenate([ei[1], loops, pad])
    items = jnp.stack([src_flat.reshape(EROWS, CB),
                       dst_flat.reshape(EROWS, CB)], axis=1)
    bhist = jnp.concatenate([bt, jnp.full((NP - N,), NG, jnp.int32)])
    btbl = jnp.concatenate([bt, jnp.zeros((NP - N,), jnp.int32)])
    xp = jnp.zeros((NP, 8), f32).at[:N, :IN_F].set(x.astype(f32))
    w1p = jnp.zeros((8, HID), f32).at[:IN_F].set(W1.astype(f32))
    zacc = jnp.zeros((NPT, 8), f32)
    zc0 = jnp.zeros((C0PT,), f32)

    mesh = plsc.VectorSubcoreMesh(core_axis_name="c", subcore_axis_name="s")
    sc_params = pltpu.CompilerParams(needs_layout_passes=False,
                                     use_tc_tiling_on_sc=False)

    deg_part, cnt_part = pl.kernel(
        _hist_body,
        compiler_params=sc_params,
        out_type=[jax.ShapeDtypeStruct((NCORE, NP), f32),
                  jax.ShapeDtypeStruct((NCORE, NGP), f32)],
        mesh=mesh,
        scratch_types=[
            pltpu.VMEM((TPW,), jnp.int32),
            pltpu.VMEM((NP,), f32),
            pltpu.VMEM((NP // NW,), jnp.int32),
            pltpu.VMEM((NGP,), f32),
            pltpu.VMEM((NSUB, NPT), f32),
            pltpu.VMEM((NPT,), f32),
            pltpu.VMEM((NSUB, NGP), f32),
            pltpu.VMEM((NGP,), f32),
            pltpu.VMEM_SHARED((NSUB, NP), f32),
            pltpu.VMEM_SHARED((NSUB, NGP), f32),
        ],
    )(dst_flat, bhist)

    dinv2, ptab2, yarr, occ2 = pl.pallas_call(
        _norm_body,
        grid=(NB1,),
        in_specs=[
            pl.BlockSpec((NCORE, KB1), lambda j: (0, j)),
            pl.BlockSpec((NCORE, NGP), lambda j: (0, 0)),
            pl.BlockSpec((KB1, 1), lambda j: (j, 0)),
            pl.BlockSpec((KB1, 8), lambda j: (j, 0)),
        ],
        out_specs=[
            pl.BlockSpec((KB1, 1), lambda j: (j, 0)),
            pl.BlockSpec((KB1, 1), lambda j: (j, 0)),
            pl.BlockSpec((KB1, 8), lambda j: (j, 0)),
            pl.BlockSpec((NG, 1), lambda j: (0, 0)),
        ],
        out_shape=[
            jax.ShapeDtypeStruct((NP, 1), f32),
            jax.ShapeDtypeStruct((NP, 1), jnp.int32),
            jax.ShapeDtypeStruct((NP, 8), f32),
            jax.ShapeDtypeStruct((NG, 1), f32),
        ],
    )(deg_part, cnt_part, btbl.reshape(NP, 1), xp)

    acc_part, c0_part = pl.kernel(
        _edge_body,
        compiler_params=sc_params,
        out_type=[jax.ShapeDtypeStruct((NCORE, NP, 8), f32),
                  jax.ShapeDtypeStruct((NCORE, C0SZ), f32)],
        mesh=mesh,
        scratch_types=[
            pltpu.VMEM((2, GRP, 2, CB), jnp.int32),
            pltpu.VMEM((NP,), jnp.int32),
            pltpu.VMEM((2, GRP, CB, 8), f32),
            pltpu.VMEM((2, GRP, CB), f32),
            pltpu.VMEM((2, GRP, CB), jnp.int32),
            pltpu.VMEM_SHARED((NP, 8), f32),
            pltpu.VMEM_SHARED((C0SZ,), f32),
            pltpu.SemaphoreType.DMA,
            pltpu.SemaphoreType.DMA,
        ],
    )(items, yarr, ptab2.reshape(NP), zacc, zc0)

    out = pl.pallas_call(
        _dense_body,
        grid=(NB2,),
        in_specs=[
            pl.BlockSpec((NCORE, KB2, 8), lambda j: (0, j, 0)),
            pl.BlockSpec((KB2, 1), lambda j: (j, 0)),
            pl.BlockSpec((NCORE, NG, KB2), lambda j: (0, 0, j)),
            pl.BlockSpec((8, HID), lambda j: (0, 0)),
            pl.BlockSpec((1, HID), lambda j: (0, 0)),
            pl.BlockSpec((HID, HID), lambda j: (0, 0)),
            pl.BlockSpec((HID, K), lambda j: (0, 0)),
            pl.BlockSpec((1, HID), lambda j: (0, 0)),
            pl.BlockSpec((1, K), lambda j: (0, 0)),
            pl.BlockSpec((NG, 1), lambda j: (0, 0)),
        ],
        out_specs=pl.BlockSpec((NG, K), lambda j: (0, 0)),
        out_shape=jax.ShapeDtypeStruct((NG, K), f32),
        scratch_shapes=[pltpu.VMEM((NG, HID), f32)],
    )(acc_part, dinv2,
      c0_part.reshape(NCORE, NG, NP),
      w1p, b1.astype(f32).reshape(1, HID), W2.astype(f32),
      Wlin.astype(f32), b2.astype(f32).reshape(1, HID),
      blin.astype(f32).reshape(1, K), occ2)
    return out


# speedup vs baseline: 1.4620x; 1.4620x over previous
import functools

import jax
import jax.numpy as jnp
from jax import lax
from jax.experimental import pallas as pl
from jax.experimental.pallas import tpu as pltpu
from jax.experimental.pallas import tpu_sc as plsc

N = 10000
NP = 10240
IN_F = 5
HID = 256
K = 122
NG = 128
NGP = 256
E = 320000
NCORE = 2
NSUB = 16
NW = NCORE * NSUB
CB = 128
NCH = 80
GRP = 8
NGRPS = NCH // GRP
TPW = NCH * CB
EPAD = TPW * NW
EROWS = EPAD // CB
NPT = NP // NSUB
C0SZ = NG * NP
C0PT = C0SZ // NSUB
HIGH = lax.Precision.HIGHEST


def _hist_body(dst_ref, bhist_ref, deg_out, cnt_out,
               buf_v, hist_v, bbuf_v, chist_v, red_v, out_v, cred_v, cout_v,
               deg_sh, cnt_sh):
    core = lax.axis_index("c")
    sid = lax.axis_index("s")
    wid = sid * NCORE + core
    z16 = jnp.zeros((16,), jnp.float32)
    o16 = jnp.ones((16,), jnp.float32)

    def zero_hist(i, c):
        hist_v[pl.ds(i * 16, 16)] = z16
        return c
    lax.fori_loop(0, NP // 16, zero_hist, 0)

    def zero_chist(i, c):
        chist_v[pl.ds(i * 16, 16)] = z16
        return c
    lax.fori_loop(0, NGP // 16, zero_chist, 0)

    pltpu.sync_copy(dst_ref.at[pl.ds(wid * TPW, TPW)], buf_v)

    def scat(i, c):
        idx = buf_v[pl.ds(i * 16, 16)]
        plsc.addupdate_scatter(hist_v, [idx], o16)
        return c
    lax.fori_loop(0, TPW // 16, scat, 0)

    bpt = NP // NW
    pltpu.sync_copy(bhist_ref.at[pl.ds(wid * bpt, bpt)], bbuf_v)

    def bscat(i, c):
        idx = bbuf_v[pl.ds(i * 16, 16)]
        plsc.addupdate_scatter(chist_v, [idx], o16)
        return c
    lax.fori_loop(0, bpt // 16, bscat, 0)

    pltpu.sync_copy(hist_v, deg_sh.at[sid])
    pltpu.sync_copy(chist_v, cnt_sh.at[sid])
    plsc.subcore_barrier()

    for r in range(NSUB):
        pltpu.sync_copy(deg_sh.at[r, pl.ds(sid * NPT, NPT)], red_v.at[r])

    def red(i, c):
        s = red_v[0, pl.ds(i * 16, 16)]
        for r in range(1, NSUB):
            s = s + red_v[r, pl.ds(i * 16, 16)]
        out_v[pl.ds(i * 16, 16)] = s
        return c
    lax.fori_loop(0, NPT // 16, red, 0)
    pltpu.sync_copy(out_v, deg_out.at[core, pl.ds(sid * NPT, NPT)])

    @pl.when(sid == 0)
    def _():
        pltpu.sync_copy(cnt_sh, cred_v)

        def cred(i, c):
            s = cred_v[0, pl.ds(i * 16, 16)]
            for r in range(1, NSUB):
                s = s + cred_v[r, pl.ds(i * 16, 16)]
            cout_v[pl.ds(i * 16, 16)] = s
            return c
        lax.fori_loop(0, NGP // 16, cred, 0)
        pltpu.sync_copy(cout_v, cnt_out.at[core])


def _edge_body(items_ref, y_ref, ptab_ref, zacc_ref, zc0_ref,
               acc_out, c0_out,
               ebuf_v, ptab_v, msg_v, wgth_v, flat_v,
               acc_sh, c0_sh, sem_g, sem_s):
    core = lax.axis_index("c")
    sid = lax.axis_index("s")
    wid = sid * NCORE + core

    pltpu.sync_copy(zacc_ref, acc_sh.at[pl.ds(sid * NPT, NPT)])
    pltpu.sync_copy(zc0_ref, c0_sh.at[pl.ds(sid * C0PT, C0PT)])

    pltpu.sync_copy(ptab_ref, ptab_v)
    plsc.subcore_barrier()

    def fire_gathers(slot, grp):
        hs = []
        for b in range(GRP):
            hs.append(pltpu.async_copy(
                y_ref.at[ebuf_v.at[slot, b, 0]], msg_v.at[slot, b], sem_g))
        return hs

    def load_idx(slot, grp):
        pltpu.sync_copy(items_ref.at[pl.ds(wid * NCH + grp * GRP, GRP)],
                        ebuf_v.at[slot])

    load_idx(0, 0)
    fire_gathers(0, 0)

    def group(g, carry):
        slot = lax.rem(g, 2)
        slot2 = 1 - slot
        for b in range(GRP):
            pltpu.make_async_copy(
                y_ref.at[ebuf_v.at[slot, b, 0]], msg_v.at[slot, b],
                sem_g).wait()
        gnext = jnp.minimum(g + 1, NGRPS - 1)
        load_idx(slot2, gnext)
        fire_gathers(slot2, gnext)
        for b in range(GRP):
            for j in range(CB // 16):
                d16 = ebuf_v[slot, b, 1, pl.ds(j * 16, 16)]
                s16 = ebuf_v[slot, b, 0, pl.ds(j * 16, 16)]
                word = plsc.load_gather(ptab_v, [d16])
                flat_v[slot, b, pl.ds(j * 16, 16)] = (
                    (word & jnp.int32(511)) * NP + s16)
                wgth_v[slot, b, pl.ds(j * 16, 16)] = plsc.bitcast(
                    word & jnp.int32(-512), jnp.float32)
        sh = []
        for b in range(GRP):
            sh.append(pltpu.async_copy(
                msg_v.at[slot, b], acc_sh.at[ebuf_v.at[slot, b, 1]],
                sem_s, add=True))
            sh.append(pltpu.async_copy(
                wgth_v.at[slot, b], c0_sh.at[flat_v.at[slot, b]],
                sem_s, add=True))
        for h in sh:
            h.wait()
        return carry
    lax.fori_loop(0, NGRPS, group, 0)

    lastslot = NGRPS % 2
    for b in range(GRP):
        pltpu.make_async_copy(
            y_ref.at[ebuf_v.at[lastslot, b, 0]], msg_v.at[lastslot, b],
            sem_g).wait()

    plsc.subcore_barrier()
    pltpu.sync_copy(acc_sh.at[pl.ds(sid * NPT, NPT)],
                    acc_out.at[core, pl.ds(sid * NPT, NPT)])
    pltpu.sync_copy(c0_sh.at[pl.ds(sid * C0PT, C0PT)],
                    c0_out.at[core, pl.ds(sid * C0PT, C0PT)])


KB1 = 2048
NB1 = NP // KB1


def _norm_body(degp, cntp, batch2, xp, dinv_o, ptab_o, wdst_o, y_o, occ_o):
    j = pl.program_id(0)
    nid0 = j * KB1 + lax.broadcasted_iota(jnp.int32, (KB1,), 0)
    deg = degp[0, :] + degp[1, :] + jnp.where(nid0 < N, 1.0, 0.0)
    dinv = jnp.where(deg > 0, lax.rsqrt(deg), 0.0).reshape(KB1, 1)
    cnt = cntp[0, :NG] + cntp[1, :NG]
    invc = 1.0 / jnp.maximum(cnt, 1.0)
    occ_o[...] = (cnt > 0).astype(jnp.float32).reshape(NG, 1)
    b = batch2[...]
    onehot = (b == lax.broadcasted_iota(jnp.int32, (KB1, NG), 1))
    invn = jnp.sum(jnp.where(onehot, invc.reshape(1, NG), 0.0),
                   axis=1, keepdims=True)
    nid = nid0.reshape(KB1, 1)
    dinv_o[...] = dinv
    wdst = jnp.where(nid < N, dinv * invn, 0.0)
    wdst_o[...] = wdst
    bits = lax.bitcast_convert_type(wdst, jnp.int32)
    ptab_o[...] = ((bits + 256) & jnp.int32(-512)) | b
    y_o[...] = xp[...] * dinv


KB2 = 2048
NB2 = NP // KB2


def _dense_body(accp, dinv_ref, y_ref, wdst_ref, batch2, c0_ref, w1_ref,
                b1_ref, w2_ref, wl_ref, b2_ref, bl_ref, occ_ref,
                out_ref, m_ref):
    j = pl.program_id(0)

    @pl.when(j == 0)
    def _():
        m_ref[...] = jnp.zeros_like(m_ref)

    acc = accp[0] + accp[1] + y_ref[...]
    dinv = dinv_ref[...]
    h1 = jnp.dot(acc * dinv, w1_ref[...], precision=HIGH) + b1_ref[...]
    r = jnp.maximum(h1, 0.0)
    nid = j * KB2 + lax.broadcasted_iota(jnp.int32, (KB2, 1), 0)
    rd = jnp.where(nid < N, r * dinv, 0.0)
    c0b = c0_ref[0] + c0_ref[1]
    m_ref[...] += jnp.dot(c0b, rd, precision=HIGH)
    onehot = (batch2[...] == lax.broadcasted_iota(jnp.int32, (KB2, NG), 1))
    m_ref[...] += lax.dot_general(
        onehot.astype(jnp.float32), wdst_ref[...] * rd,
        (((0,), (0,)), ((), ())), precision=HIGH)

    @pl.when(j == NB2 - 1)
    def _():
        w2l = jnp.dot(w2_ref[...], wl_ref[...], precision=HIGH)
        bl2 = jnp.dot(b2_ref[...], wl_ref[...], precision=HIGH)
        out_ref[...] = (jnp.dot(m_ref[...], w2l, precision=HIGH)
                        + occ_ref[...] * bl2 + bl_ref[...])


def kernel(x, edge_index, batch, W1, b1, W2, b2, Wlin, blin):
    f32 = jnp.float32
    ei = edge_index.astype(jnp.int32)
    bt = batch.astype(jnp.int32)
    pad = jnp.full((EPAD - E,), N, jnp.int32)
    src_flat = jnp.concatenate([ei[0], pad])
    dst_flat = jnp.concatenate([ei[1], pad])
    items = jnp.stack([src_flat.reshape(EROWS, CB),
                       dst_flat.reshape(EROWS, CB)], axis=1)
    bhist = jnp.concatenate([bt, jnp.full((NP - N,), NG, jnp.int32)])
    btbl = jnp.concatenate([bt, jnp.zeros((NP - N,), jnp.int32)])
    xp = jnp.zeros((NP, 8), f32).at[:N, :IN_F].set(x.astype(f32))
    w1p = jnp.zeros((8, HID), f32).at[:IN_F].set(W1.astype(f32))
    zacc = jnp.zeros((NPT, 8), f32)
    zc0 = jnp.zeros((C0PT,), f32)

    mesh = plsc.VectorSubcoreMesh(core_axis_name="c", subcore_axis_name="s")
    sc_params = pltpu.CompilerParams(needs_layout_passes=False,
                                     use_tc_tiling_on_sc=False)

    deg_part, cnt_part = pl.kernel(
        _hist_body,
        compiler_params=sc_params,
        out_type=[jax.ShapeDtypeStruct((NCORE, NP), f32),
                  jax.ShapeDtypeStruct((NCORE, NGP), f32)],
        mesh=mesh,
        scratch_types=[
            pltpu.VMEM((TPW,), jnp.int32),
            pltpu.VMEM((NP,), f32),
            pltpu.VMEM((NP // NW,), jnp.int32),
            pltpu.VMEM((NGP,), f32),
            pltpu.VMEM((NSUB, NPT), f32),
            pltpu.VMEM((NPT,), f32),
            pltpu.VMEM((NSUB, NGP), f32),
            pltpu.VMEM((NGP,), f32),
            pltpu.VMEM_SHARED((NSUB, NP), f32),
            pltpu.VMEM_SHARED((NSUB, NGP), f32),
        ],
    )(dst_flat, bhist)

    dinv2, ptab2, wdst2, yarr, occ2 = pl.pallas_call(
        _norm_body,
        grid=(NB1,),
        in_specs=[
            pl.BlockSpec((NCORE, KB1), lambda j: (0, j)),
            pl.BlockSpec((NCORE, NGP), lambda j: (0, 0)),
            pl.BlockSpec((KB1, 1), lambda j: (j, 0)),
            pl.BlockSpec((KB1, 8), lambda j: (j, 0)),
        ],
        out_specs=[
            pl.BlockSpec((KB1, 1), lambda j: (j, 0)),
            pl.BlockSpec((KB1, 1), lambda j: (j, 0)),
            pl.BlockSpec((KB1, 1), lambda j: (j, 0)),
            pl.BlockSpec((KB1, 8), lambda j: (j, 0)),
            pl.BlockSpec((NG, 1), lambda j: (0, 0)),
        ],
        out_shape=[
            jax.ShapeDtypeStruct((NP, 1), f32),
            jax.ShapeDtypeStruct((NP, 1), jnp.int32),
            jax.ShapeDtypeStruct((NP, 1), f32),
            jax.ShapeDtypeStruct((NP, 8), f32),
            jax.ShapeDtypeStruct((NG, 1), f32),
        ],
    )(deg_part, cnt_part, btbl.reshape(NP, 1), xp)

    acc_part, c0_part = pl.kernel(
        _edge_body,
        compiler_params=sc_params,
        out_type=[jax.ShapeDtypeStruct((NCORE, NP, 8), f32),
                  jax.ShapeDtypeStruct((NCORE, C0SZ), f32)],
        mesh=mesh,
        scratch_types=[
            pltpu.VMEM((2, GRP, 2, CB), jnp.int32),
            pltpu.VMEM((NP,), jnp.int32),
            pltpu.VMEM((2, GRP, CB, 8), f32),
            pltpu.VMEM((2, GRP, CB), f32),
            pltpu.VMEM((2, GRP, CB), jnp.int32),
            pltpu.VMEM_SHARED((NP, 8), f32),
            pltpu.VMEM_SHARED((C0SZ,), f32),
            pltpu.SemaphoreType.DMA,
            pltpu.SemaphoreType.DMA,
        ],
    )(items, yarr, ptab2.reshape(NP), zacc, zc0)

    out = pl.pallas_call(
        _dense_body,
        grid=(NB2,),
        in_specs=[
            pl.BlockSpec((NCORE, KB2, 8), lambda j: (0, j, 0)),
            pl.BlockSpec((KB2, 1), lambda j: (j, 0)),
            pl.BlockSpec((KB2, 8), lambda j: (j, 0)),
            pl.BlockSpec((KB2, 1), lambda j: (j, 0)),
            pl.BlockSpec((KB2, 1), lambda j: (j, 0)),
            pl.BlockSpec((NCORE, NG, KB2), lambda j: (0, 0, j)),
            pl.BlockSpec((8, HID), lambda j: (0, 0)),
            pl.BlockSpec((1, HID), lambda j: (0, 0)),
            pl.BlockSpec((HID, HID), lambda j: (0, 0)),
            pl.BlockSpec((HID, K), lambda j: (0, 0)),
            pl.BlockSpec((1, HID), lambda j: (0, 0)),
            pl.BlockSpec((1, K), lambda j: (0, 0)),
            pl.BlockSpec((NG, 1), lambda j: (0, 0)),
        ],
        out_specs=pl.BlockSpec((NG, K), lambda j: (0, 0)),
        out_shape=jax.ShapeDtypeStruct((NG, K), f32),
        scratch_shapes=[pltpu.VMEM((NG, HID), f32)],
    )(acc_part, dinv2, yarr, wdst2, btbl.reshape(NP, 1),
      c0_part.reshape(NCORE, NG, NP),
      w1p, b1.astype(f32).reshape(1, HID), W2.astype(f32),
      Wlin.astype(f32), b2.astype(f32).reshape(1, HID),
      blin.astype(f32).reshape(1, K), occ2)
    return out


# speedup vs baseline: 1.7705x; 1.2110x over previous
import functools

import jax
import jax.numpy as jnp
from jax import lax
from jax.experimental import pallas as pl
from jax.experimental.pallas import tpu as pltpu
from jax.experimental.pallas import tpu_sc as plsc

N = 10000
NP = 10240
IN_F = 5
HID = 256
K = 122
NG = 128
NGP = 256
E = 320000
NCORE = 2
NSUB = 16
NW = NCORE * NSUB
CB = 128
NCH = 80
GRP = 10
NGRPS = NCH // GRP
TPW = NCH * CB
EPAD = TPW * NW
EROWS = EPAD // CB
NPT = NP // NSUB
C0SZ = NG * NP
C0PT = C0SZ // NSUB
HIGH = lax.Precision.HIGHEST


def _hist_body(dst_ref, bhist_ref, deg_out, cnt_out,
               buf_v, hist_v, bbuf_v, chist_v, red_v, out_v, cred_v, cout_v,
               deg_sh, cnt_sh):
    core = lax.axis_index("c")
    sid = lax.axis_index("s")
    wid = sid * NCORE + core
    z16 = jnp.zeros((16,), jnp.float32)
    o16 = jnp.ones((16,), jnp.float32)

    def zero_hist(i, c):
        hist_v[pl.ds(i * 16, 16)] = z16
        return c
    lax.fori_loop(0, NP // 16, zero_hist, 0)

    def zero_chist(i, c):
        chist_v[pl.ds(i * 16, 16)] = z16
        return c
    lax.fori_loop(0, NGP // 16, zero_chist, 0)

    pltpu.sync_copy(dst_ref.at[pl.ds(wid * TPW, TPW)], buf_v)

    def scat(i, c):
        idx = buf_v[pl.ds(i * 16, 16)]
        plsc.addupdate_scatter(hist_v, [idx], o16)
        return c
    lax.fori_loop(0, TPW // 16, scat, 0)

    bpt = NP // NW
    pltpu.sync_copy(bhist_ref.at[pl.ds(wid * bpt, bpt)], bbuf_v)

    def bscat(i, c):
        idx = bbuf_v[pl.ds(i * 16, 16)]
        plsc.addupdate_scatter(chist_v, [idx], o16)
        return c
    lax.fori_loop(0, bpt // 16, bscat, 0)

    pltpu.sync_copy(hist_v, deg_sh.at[sid])
    pltpu.sync_copy(chist_v, cnt_sh.at[sid])
    plsc.subcore_barrier()

    for r in range(NSUB):
        pltpu.sync_copy(deg_sh.at[r, pl.ds(sid * NPT, NPT)], red_v.at[r])

    def red(i, c):
        s = red_v[0, pl.ds(i * 16, 16)]
        for r in range(1, NSUB):
            s = s + red_v[r, pl.ds(i * 16, 16)]
        out_v[pl.ds(i * 16, 16)] = s
        return c
    lax.fori_loop(0, NPT // 16, red, 0)
    pltpu.sync_copy(out_v, deg_out.at[core, pl.ds(sid * NPT, NPT)])

    @pl.when(sid == 0)
    def _():
        pltpu.sync_copy(cnt_sh, cred_v)

        def cred(i, c):
            s = cred_v[0, pl.ds(i * 16, 16)]
            for r in range(1, NSUB):
                s = s + cred_v[r, pl.ds(i * 16, 16)]
            cout_v[pl.ds(i * 16, 16)] = s
            return c
        lax.fori_loop(0, NGP // 16, cred, 0)
        pltpu.sync_copy(cout_v, cnt_out.at[core])


def _edge_body(items_ref, y_ref, ptab_ref, zacc_ref, zc0_ref,
               acc_out, c0_out,
               ebuf_v, ptab_v, msg_v, wgth_v, flat_v,
               acc_sh, c0_sh, sem_g, sem_s):
    core = lax.axis_index("c")
    sid = lax.axis_index("s")
    wid = sid * NCORE + core

    pltpu.sync_copy(zacc_ref, acc_sh.at[pl.ds(sid * NPT, NPT)])
    pltpu.sync_copy(zc0_ref, c0_sh.at[pl.ds(sid * C0PT, C0PT)])

    pltpu.sync_copy(ptab_ref, ptab_v)
    plsc.subcore_barrier()

    def fire_gathers(slot, grp):
        hs = []
        for b in range(GRP):
            hs.append(pltpu.async_copy(
                y_ref.at[ebuf_v.at[slot, b, 0]], msg_v.at[slot, b], sem_g))
        return hs

    def load_idx(slot, grp):
        pltpu.sync_copy(items_ref.at[pl.ds(wid * NCH + grp * GRP, GRP)],
                        ebuf_v.at[slot])

    load_idx(0, 0)
    fire_gathers(0, 0)

    def group(g, carry):
        slot = lax.rem(g, 2)
        slot2 = 1 - slot
        for b in range(GRP):
            pltpu.make_async_copy(
                y_ref.at[ebuf_v.at[slot, b, 0]], msg_v.at[slot, b],
                sem_g).wait()
        gnext = jnp.minimum(g + 1, NGRPS - 1)
        load_idx(slot2, gnext)
        fire_gathers(slot2, gnext)
        for b in range(GRP):
            for j in range(CB // 16):
                d16 = ebuf_v[slot, b, 1, pl.ds(j * 16, 16)]
                s16 = ebuf_v[slot, b, 0, pl.ds(j * 16, 16)]
                word = plsc.load_gather(ptab_v, [d16])
                flat_v[slot, b, pl.ds(j * 16, 16)] = (
                    (word & jnp.int32(511)) * NP + s16)
                wgth_v[slot, b, pl.ds(j * 16, 16)] = plsc.bitcast(
                    word & jnp.int32(-512), jnp.float32)
        sh = []
        for b in range(GRP):
            sh.append(pltpu.async_copy(
                msg_v.at[slot, b], acc_sh.at[ebuf_v.at[slot, b, 1]],
                sem_s, add=True))
            sh.append(pltpu.async_copy(
                wgth_v.at[slot, b], c0_sh.at[flat_v.at[slot, b]],
                sem_s, add=True))
        for h in sh:
            h.wait()
        return carry
    lax.fori_loop(0, NGRPS, group, 0)

    lastslot = NGRPS % 2
    for b in range(GRP):
        pltpu.make_async_copy(
            y_ref.at[ebuf_v.at[lastslot, b, 0]], msg_v.at[lastslot, b],
            sem_g).wait()

    plsc.subcore_barrier()
    pltpu.sync_copy(acc_sh.at[pl.ds(sid * NPT, NPT)],
                    acc_out.at[core, pl.ds(sid * NPT, NPT)])
    pltpu.sync_copy(c0_sh.at[pl.ds(sid * C0PT, C0PT)],
                    c0_out.at[core, pl.ds(sid * C0PT, C0PT)])


KB1 = 2048
NB1 = NP // KB1


def _norm_body(degp, cntp, batch2, xp, dinv_o, ptab_o, wdst_o, y_o, occ_o):
    j = pl.program_id(0)
    nid0 = j * KB1 + lax.broadcasted_iota(jnp.int32, (KB1,), 0)
    deg = degp[0, :] + degp[1, :] + jnp.where(nid0 < N, 1.0, 0.0)
    dinv = jnp.where(deg > 0, lax.rsqrt(deg), 0.0).reshape(KB1, 1)
    cnt = cntp[0, :NG] + cntp[1, :NG]
    invc = 1.0 / jnp.maximum(cnt, 1.0)
    occ_o[...] = (cnt > 0).astype(jnp.float32).reshape(NG, 1)
    b = batch2[...]
    onehot = (b == lax.broadcasted_iota(jnp.int32, (KB1, NG), 1))
    invn = jnp.sum(jnp.where(onehot, invc.reshape(1, NG), 0.0),
                   axis=1, keepdims=True)
    nid = nid0.reshape(KB1, 1)
    dinv_o[...] = dinv
    wdst = jnp.where(nid < N, dinv * invn, 0.0)
    wdst_o[...] = wdst
    bits = lax.bitcast_convert_type(wdst, jnp.int32)
    ptab_o[...] = ((bits + 256) & jnp.int32(-512)) | b
    y_o[...] = xp[...] * dinv


KB2 = 2048
NB2 = NP // KB2


def _dense_body(accp, dinv_ref, y_ref, wdst_ref, batch2, c0_ref, w1_ref,
                b1_ref, w2_ref, wl_ref, b2_ref, bl_ref, occ_ref,
                out_ref, m_ref):
    j = pl.program_id(0)

    @pl.when(j == 0)
    def _():
        m_ref[...] = jnp.zeros_like(m_ref)

    acc = accp[0] + accp[1] + y_ref[...]
    dinv = dinv_ref[...]
    h1 = jnp.dot(acc * dinv, w1_ref[...], precision=HIGH) + b1_ref[...]
    r = jnp.maximum(h1, 0.0)
    nid = j * KB2 + lax.broadcasted_iota(jnp.int32, (KB2, 1), 0)
    rd = jnp.where(nid < N, r * dinv, 0.0)
    c0b = c0_ref[0] + c0_ref[1]
    m_ref[...] += jnp.dot(c0b, rd, precision=HIGH)
    onehot = (batch2[...] == lax.broadcasted_iota(jnp.int32, (KB2, NG), 1))
    m_ref[...] += lax.dot_general(
        onehot.astype(jnp.float32), wdst_ref[...] * rd,
        (((0,), (0,)), ((), ())), precision=HIGH)

    @pl.when(j == NB2 - 1)
    def _():
        w2l = jnp.dot(w2_ref[...], wl_ref[...], precision=HIGH)
        bl2 = jnp.dot(b2_ref[...], wl_ref[...], precision=HIGH)
        out_ref[...] = (jnp.dot(m_ref[...], w2l, precision=HIGH)
                        + occ_ref[...] * bl2 + bl_ref[...])


def kernel(x, edge_index, batch, W1, b1, W2, b2, Wlin, blin):
    f32 = jnp.float32
    ei = edge_index.astype(jnp.int32)
    bt = batch.astype(jnp.int32)
    pad = N + jnp.arange(EPAD - E, dtype=jnp.int32) % (NP - N)
    src_flat = jnp.concatenate([ei[0], pad])
    dst_flat = jnp.concatenate([ei[1], pad])
    items = jnp.stack([src_flat.reshape(EROWS, CB),
                       dst_flat.reshape(EROWS, CB)], axis=1)
    bhist = jnp.concatenate([bt, jnp.full((NP - N,), NG, jnp.int32)])
    btbl = jnp.concatenate([bt, jnp.zeros((NP - N,), jnp.int32)])
    xp = jnp.zeros((NP, 8), f32).at[:N, :IN_F].set(x.astype(f32))
    w1p = jnp.zeros((8, HID), f32).at[:IN_F].set(W1.astype(f32))
    zacc = jnp.zeros((NPT, 8), f32)
    zc0 = jnp.zeros((C0PT,), f32)

    mesh = plsc.VectorSubcoreMesh(core_axis_name="c", subcore_axis_name="s")
    sc_params = pltpu.CompilerParams(needs_layout_passes=False,
                                     use_tc_tiling_on_sc=False)

    deg_part, cnt_part = pl.kernel(
        _hist_body,
        compiler_params=sc_params,
        out_type=[jax.ShapeDtypeStruct((NCORE, NP), f32),
                  jax.ShapeDtypeStruct((NCORE, NGP), f32)],
        mesh=mesh,
        scratch_types=[
            pltpu.VMEM((TPW,), jnp.int32),
            pltpu.VMEM((NP,), f32),
            pltpu.VMEM((NP // NW,), jnp.int32),
            pltpu.VMEM((NGP,), f32),
            pltpu.VMEM((NSUB, NPT), f32),
            pltpu.VMEM((NPT,), f32),
            pltpu.VMEM((NSUB, NGP), f32),
            pltpu.VMEM((NGP,), f32),
            pltpu.VMEM_SHARED((NSUB, NP), f32),
            pltpu.VMEM_SHARED((NSUB, NGP), f32),
        ],
    )(dst_flat, bhist)

    dinv2, ptab2, wdst2, yarr, occ2 = pl.pallas_call(
        _norm_body,
        grid=(NB1,),
        in_specs=[
            pl.BlockSpec((NCORE, KB1), lambda j: (0, j)),
            pl.BlockSpec((NCORE, NGP), lambda j: (0, 0)),
            pl.BlockSpec((KB1, 1), lambda j: (j, 0)),
            pl.BlockSpec((KB1, 8), lambda j: (j, 0)),
        ],
        out_specs=[
            pl.BlockSpec((KB1, 1), lambda j: (j, 0)),
            pl.BlockSpec((KB1, 1), lambda j: (j, 0)),
            pl.BlockSpec((KB1, 1), lambda j: (j, 0)),
            pl.BlockSpec((KB1, 8), lambda j: (j, 0)),
            pl.BlockSpec((NG, 1), lambda j: (0, 0)),
        ],
        out_shape=[
            jax.ShapeDtypeStruct((NP, 1), f32),
            jax.ShapeDtypeStruct((NP, 1), jnp.int32),
            jax.ShapeDtypeStruct((NP, 1), f32),
            jax.ShapeDtypeStruct((NP, 8), f32),
            jax.ShapeDtypeStruct((NG, 1), f32),
        ],
    )(deg_part, cnt_part, btbl.reshape(NP, 1), xp)

    acc_part, c0_part = pl.kernel(
        _edge_body,
        compiler_params=sc_params,
        out_type=[jax.ShapeDtypeStruct((NCORE, NP, 8), f32),
                  jax.ShapeDtypeStruct((NCORE, C0SZ), f32)],
        mesh=mesh,
        scratch_types=[
            pltpu.VMEM((2, GRP, 2, CB), jnp.int32),
            pltpu.VMEM((NP,), jnp.int32),
            pltpu.VMEM((2, GRP, CB, 8), f32),
            pltpu.VMEM((2, GRP, CB), f32),
            pltpu.VMEM((2, GRP, CB), jnp.int32),
            pltpu.VMEM_SHARED((NP, 8), f32),
            pltpu.VMEM_SHARED((C0SZ,), f32),
            pltpu.SemaphoreType.DMA,
            pltpu.SemaphoreType.DMA,
        ],
    )(items, yarr, ptab2.reshape(NP), zacc, zc0)

    out = pl.pallas_call(
        _dense_body,
        grid=(NB2,),
        in_specs=[
            pl.BlockSpec((NCORE, KB2, 8), lambda j: (0, j, 0)),
            pl.BlockSpec((KB2, 1), lambda j: (j, 0)),
            pl.BlockSpec((KB2, 8), lambda j: (j, 0)),
            pl.BlockSpec((KB2, 1), lambda j: (j, 0)),
            pl.BlockSpec((KB2, 1), lambda j: (j, 0)),
            pl.BlockSpec((NCORE, NG, KB2), lambda j: (0, 0, j)),
            pl.BlockSpec((8, HID), lambda j: (0, 0)),
            pl.BlockSpec((1, HID), lambda j: (0, 0)),
            pl.BlockSpec((HID, HID), lambda j: (0, 0)),
            pl.BlockSpec((HID, K), lambda j: (0, 0)),
            pl.BlockSpec((1, HID), lambda j: (0, 0)),
            pl.BlockSpec((1, K), lambda j: (0, 0)),
            pl.BlockSpec((NG, 1), lambda j: (0, 0)),
        ],
        out_specs=pl.BlockSpec((NG, K), lambda j: (0, 0)),
        out_shape=jax.ShapeDtypeStruct((NG, K), f32),
        scratch_shapes=[pltpu.VMEM((NG, HID), f32)],
    )(acc_part, dinv2, yarr, wdst2, btbl.reshape(NP, 1),
      c0_part.reshape(NCORE, NG, NP),
      w1p, b1.astype(f32).reshape(1, HID), W2.astype(f32),
      Wlin.astype(f32), b2.astype(f32).reshape(1, HID),
      blin.astype(f32).reshape(1, K), occ2)
    return out
